# baseline (speedup 1.0000x reference)
import functools

import jax
import jax.numpy as jnp
from jax import lax
from jax.experimental import pallas as pl
from jax.experimental.pallas import tpu as pltpu
from jax.experimental.pallas import tpu_sc as plsc

N = 10000
E = 320000
D = 128

NW = 32
CK = 128
CH = 80
PE = NW * CH * CK
NP = 10240
RPT = NP // 16


NB = 2
NI = 4


@functools.cache
def _make_sc_segment():
    mesh = plsc.VectorSubcoreMesh(core_axis_name="c", subcore_axis_name="s")
    return pl.kernel(
        _sc_segment_body,
        out_type=jax.ShapeDtypeStruct((2, NP, D), jnp.float32),
        mesh=mesh,
        scratch_types=[
            [pltpu.VMEM((2, CK), jnp.int32) for _ in range(NI)],
            [pltpu.VMEM((CK, D), jnp.float32) for _ in range(NB)],
            pltpu.VMEM_SHARED((NP, D), jnp.float32),
            [pltpu.SemaphoreType.DMA for _ in range(NI)],
            [pltpu.SemaphoreType.DMA for _ in range(NB)],
            [pltpu.SemaphoreType.DMA for _ in range(NB)],
        ],
    )


def _sc_segment_body(h_hbm, eidx_hbm, out_hbm, idxs, rows, acc,
                     isems, gsems, ssems):
    c = lax.axis_index("c")
    s = lax.axis_index("s")
    wid = (1 - c) * 16 + s
    pltpu.sync_copy(h_hbm.at[pl.ds(s * RPT, RPT)], acc.at[pl.ds(s * RPT, RPT)])
    plsc.subcore_barrier()

    def idx_issue(j, slot):
        pltpu.async_copy(eidx_hbm.at[wid, j], idxs[slot], isems[slot])

    def gather(j, islot, b):
        return pltpu.make_async_copy(
            h_hbm.at[idxs[islot].at[0]], rows[b], gsems[b])

    def scatter(islot, b):
        return pltpu.make_async_copy(
            rows[b], acc.at[idxs[islot].at[1]], ssems[b])

    idx_issue(0, 0)
    idx_issue(1, 1)
    pltpu.make_async_copy(eidx_hbm.at[wid, 0], idxs[0], isems[0]).wait()
    gather(0, 0, 0).start()

    def body(g, carry):
        for k in range(4):
            j = g * 4 + k
            b = k % 2
            o = 1 - b
            kn1 = (k + 1) % 4
            kn2 = (k + 2) % 4

            @pl.when(j >= 1)
            def _():
                scatter(kn1, o).wait()

            @pl.when(j + 1 < CH)
            def _():
                pltpu.make_async_copy(
                    eidx_hbm.at[wid, j + 1], idxs[kn1], isems[kn1]).wait()
                gather(j + 1, kn1, o).start()

            @pl.when(j + 2 < CH)
            def _():
                idx_issue(j + 2, kn2)

            gather(j, k, b).wait()
            scatter(k, b).start(add=True)
        return carry

    lax.fori_loop(0, CH // 4, body, 0)
    pltpu.make_async_copy(
        rows[(CH - 1) % 2], acc.at[idxs[(CH - 1) % 4].at[1]],
        ssems[(CH - 1) % 2]).wait()
    plsc.subcore_barrier()
    pltpu.sync_copy(acc.at[pl.ds(s * RPT, RPT)], out_hbm.at[c, pl.ds(s * RPT, RPT)])


def _dense_body(p0, p1, h, w1, g1, b1, w2, g2, b2, xo, pool):
    mask = lax.broadcasted_iota(jnp.int32, (NP, 1), 0) < N
    x = jnp.where(mask, p0[...] + p1[...] - h[...], 0.0)
    t = jnp.dot(x, w1[...], preferred_element_type=jnp.float32)
    mu = jnp.sum(t, axis=0, keepdims=True) * (1.0 / N)
    d = jnp.where(mask, t - mu, 0.0)
    var = jnp.sum(d * d, axis=0, keepdims=True) * (1.0 / N)
    y = g1[...] * d * jax.lax.rsqrt(var + 1e-5) + b1[...]
    y = jnp.where(mask, jnp.maximum(y, 0.0), 0.0)
    u = jnp.dot(y, w2[...], preferred_element_type=jnp.float32)
    mu2 = jnp.sum(u, axis=0, keepdims=True) * (1.0 / N)
    d2 = jnp.where(mask, u - mu2, 0.0)
    var2 = jnp.sum(d2 * d2, axis=0, keepdims=True) * (1.0 / N)
    z = g2[...] * d2 * jax.lax.rsqrt(var2 + 1e-5) + b2[...]
    z = jnp.where(mask, jnp.maximum(z, 0.0), 0.0)
    xo[...] = z
    pool[...] = jnp.sum(z, axis=0, keepdims=True)


_dense = pl.pallas_call(
    _dense_body,
    out_shape=(
        jax.ShapeDtypeStruct((NP, D), jnp.float32),
        jax.ShapeDtypeStruct((1, D), jnp.float32),
    ),
)


def _layer(h_pad, eidx, W1, g1, b1, W2, bng, bnb):
    p = _make_sc_segment()(h_pad, eidx)
    return _dense(p[0], p[1], h_pad,
                  W1, g1.reshape(1, D), b1.reshape(1, D),
                  W2, bng.reshape(1, D), bnb.reshape(1, D))


def kernel(h, edge_index, W1_0, g1_0, b1_0, W2_0, bng_0, bnb_0,
           W1_1, g1_1, b1_1, W2_1, bng_1, bnb_1):
    pad = PE - E
    src3 = jnp.concatenate(
        [edge_index[0], jnp.zeros((pad,), jnp.int32)]).reshape(NW, CH, CK)
    trash = N + (jnp.arange(pad, dtype=jnp.int32) % (NP - N))
    dst3 = jnp.concatenate([edge_index[1], trash]).reshape(NW, CH, CK)
    eidx = jnp.stack([src3, dst3], axis=2)
    h_pad = jnp.pad(h, ((0, NP - N), (0, 0)))
    h1, p0 = _layer(h_pad, eidx, W1_0, g1_0, b1_0, W2_0, bng_0, bnb_0)
    h2, p1 = _layer(h1, eidx, W1_1, g1_1, b1_1, W2_1, bng_1, bnb_1)
    return h2[:N], jnp.concatenate([p0, p1], axis=1)


# speedup vs baseline: 3.1854x; 3.1854x over previous
import functools

import jax
import jax.numpy as jnp
from jax import lax
from jax.experimental import pallas as pl
from jax.experimental.pallas import tpu as pltpu
from jax.experimental.pallas import tpu_sc as plsc

N = 10000
E = 320000
D = 128

NW = 32
CK = 128
CH = 80
PE = NW * CH * CK
NP = 10240
RPT = NP // 16


NB = 2
NI = 4


@functools.cache
def _make_sc_segment():
    mesh = plsc.VectorSubcoreMesh(core_axis_name="c", subcore_axis_name="s")
    return pl.kernel(
        _sc_segment_body,
        out_type=jax.ShapeDtypeStruct((2, NP, D), jnp.float32),
        mesh=mesh,
        scratch_types=[
            [pltpu.VMEM((2, CK), jnp.int32) for _ in range(NI)],
            [pltpu.VMEM((CK, D), jnp.float32) for _ in range(NB)],
            pltpu.VMEM_SHARED((NP, D), jnp.float32),
            [pltpu.SemaphoreType.DMA for _ in range(NI)],
            [pltpu.SemaphoreType.DMA for _ in range(NB)],
            [pltpu.SemaphoreType.DMA for _ in range(NB)],
        ],
    )


def _sc_segment_body(h_hbm, eidx_hbm, out_hbm, idxs, rows, acc,
                     isems, gsems, ssems):
    c = lax.axis_index("c")
    s = lax.axis_index("s")
    wid = c * 16 + s
    pltpu.sync_copy(h_hbm.at[pl.ds(s * RPT, RPT)], acc.at[pl.ds(s * RPT, RPT)])
    plsc.subcore_barrier()

    def idx_issue(j, slot):
        pltpu.async_copy(eidx_hbm.at[wid, j], idxs[slot], isems[slot])

    def gather(j, islot, b):
        return pltpu.make_async_copy(
            h_hbm.at[idxs[islot].at[0]], rows[b], gsems[b])

    def scatter(islot, b):
        return pltpu.make_async_copy(
            rows[b], acc.at[idxs[islot].at[1]], ssems[b])

    idx_issue(0, 0)
    idx_issue(1, 1)
    pltpu.make_async_copy(eidx_hbm.at[wid, 0], idxs[0], isems[0]).wait()
    gather(0, 0, 0).start()

    def body(g, carry):
        for k in range(4):
            j = g * 4 + k
            b = k % 2
            o = 1 - b
            kn1 = (k + 1) % 4
            kn2 = (k + 2) % 4

            @pl.when(j >= 1)
            def _():
                scatter(kn1, o).wait()

            @pl.when(j + 1 < CH)
            def _():
                pltpu.make_async_copy(
                    eidx_hbm.at[wid, j + 1], idxs[kn1], isems[kn1]).wait()
                gather(j + 1, kn1, o).start()

            @pl.when(j + 2 < CH)
            def _():
                idx_issue(j + 2, kn2)

            gather(j, k, b).wait()
            scatter(k, b).start(add=True)
        return carry

    lax.fori_loop(0, CH // 4, body, 0)
    pltpu.make_async_copy(
        rows[(CH - 1) % 2], acc.at[idxs[(CH - 1) % 4].at[1]],
        ssems[(CH - 1) % 2]).wait()
    plsc.subcore_barrier()
    pltpu.sync_copy(acc.at[pl.ds(s * RPT, RPT)], out_hbm.at[c, pl.ds(s * RPT, RPT)])


def _dense_body(p0, p1, h, w1, g1, b1, w2, g2, b2, xo, pool):
    mask = lax.broadcasted_iota(jnp.int32, (NP, 1), 0) < N
    x = jnp.where(mask, p0[...] + p1[...] - h[...], 0.0)
    t = jnp.dot(x, w1[...], preferred_element_type=jnp.float32)
    mu = jnp.sum(t, axis=0, keepdims=True) * (1.0 / N)
    d = jnp.where(mask, t - mu, 0.0)
    var = jnp.sum(d * d, axis=0, keepdims=True) * (1.0 / N)
    y = g1[...] * d * jax.lax.rsqrt(var + 1e-5) + b1[...]
    y = jnp.where(mask, jnp.maximum(y, 0.0), 0.0)
    u = jnp.dot(y, w2[...], preferred_element_type=jnp.float32)
    mu2 = jnp.sum(u, axis=0, keepdims=True) * (1.0 / N)
    d2 = jnp.where(mask, u - mu2, 0.0)
    var2 = jnp.sum(d2 * d2, axis=0, keepdims=True) * (1.0 / N)
    z = g2[...] * d2 * jax.lax.rsqrt(var2 + 1e-5) + b2[...]
    z = jnp.where(mask, jnp.maximum(z, 0.0), 0.0)
    xo[...] = z
    pool[...] = jnp.sum(z, axis=0, keepdims=True)


_dense = pl.pallas_call(
    _dense_body,
    out_shape=(
        jax.ShapeDtypeStruct((NP, D), jnp.float32),
        jax.ShapeDtypeStruct((1, D), jnp.float32),
    ),
)


def _layer(h_pad, eidx, W1, g1, b1, W2, bng, bnb):
    p = _make_sc_segment()(h_pad, eidx)
    return _dense(p[0], p[1], h_pad,
                  W1, g1.reshape(1, D), b1.reshape(1, D),
                  W2, bng.reshape(1, D), bnb.reshape(1, D))


def kernel(h, edge_index, W1_0, g1_0, b1_0, W2_0, bng_0, bnb_0,
           W1_1, g1_1, b1_1, W2_1, bng_1, bnb_1):
    pad = PE - E
    ar = jnp.arange(pad, dtype=jnp.int32)
    src3 = jnp.concatenate(
        [edge_index[0], (ar * 97) % N]).reshape(CH, NW, CK).transpose(1, 0, 2)
    trash = N + (ar * 7) % (NP - N)
    dst3 = jnp.concatenate(
        [edge_index[1], trash]).reshape(CH, NW, CK).transpose(1, 0, 2)
    eidx = jnp.stack([src3, dst3], axis=2)
    h_pad = jnp.pad(h, ((0, NP - N), (0, 0)))
    h1, p0 = _layer(h_pad, eidx, W1_0, g1_0, b1_0, W2_0, bng_0, bnb_0)
    h2, p1 = _layer(h1, eidx, W1_1, g1_1, b1_1, W2_1, bng_1, bnb_1)
    return h2[:N], jnp.concatenate([p0, p1], axis=1)


# speedup vs baseline: 3.6733x; 1.1532x over previous
import functools

import jax
import jax.numpy as jnp
from jax import lax
from jax.experimental import pallas as pl
from jax.experimental.pallas import tpu as pltpu
from jax.experimental.pallas import tpu_sc as plsc

N = 10000
E = 320000
D = 128

NW = 32
CK = 128
CH = 80
NP = 10240
RPT = NP // 16

FULL = E // CK
NTLO = FULL // NW
NTREM = FULL % NW


NI = 4


@functools.cache
def _make_sc_segment():
    mesh = plsc.VectorSubcoreMesh(core_axis_name="c", subcore_axis_name="s")
    return pl.kernel(
        _sc_segment_body,
        out_type=jax.ShapeDtypeStruct((2, NP, D), jnp.float32),
        mesh=mesh,
        scratch_types=[
            [pltpu.VMEM((2, CK), jnp.int32) for _ in range(NI)],
            [pltpu.VMEM((CK, D), jnp.float32) for _ in range(2)],
            pltpu.VMEM_SHARED((NP, D), jnp.float32),
            [pltpu.SemaphoreType.DMA for _ in range(NI)],
            [pltpu.SemaphoreType.DMA for _ in range(2)],
            [pltpu.SemaphoreType.DMA for _ in range(2)],
        ],
    )


def _sc_segment_body(h_hbm, eidx_hbm, out_hbm, idxs, rows, acc,
                     isems, gsems, ssems):
    c = lax.axis_index("c")
    s = lax.axis_index("s")
    wid = c * 16 + s
    nt = jnp.where(wid < NTREM, NTLO + 1, NTLO)
    pltpu.sync_copy(h_hbm.at[pl.ds(s * RPT, RPT)], acc.at[pl.ds(s * RPT, RPT)])
    plsc.subcore_barrier()

    def idx_copy(j, slot, row):
        return pltpu.make_async_copy(
            eidx_hbm.at[row, pl.ds(j * (NW * CK) + wid * CK, CK)],
            idxs[slot].at[row], isems[slot])

    def idx_issue(j, slot):
        idx_copy(j, slot, 0).start()
        idx_copy(j, slot, 1).start()

    def idx_wait(j, slot):
        idx_copy(j, slot, 0).wait()
        idx_copy(j, slot, 1).wait()

    def gather(islot, b):
        return pltpu.make_async_copy(
            h_hbm.at[idxs[islot].at[0]], rows[b], gsems[b])

    def scatter(islot, b):
        return pltpu.make_async_copy(
            rows[b], acc.at[idxs[islot].at[1]], ssems[b])

    idx_issue(0, 0)
    idx_issue(1, 1)
    idx_wait(0, 0)
    gather(0, 0).start()

    def body(g, carry):
        for k in range(4):
            j = g * 4 + k
            b = k % 2
            o = 1 - b
            kn1 = (k + 1) % 4
            kn2 = (k + 2) % 4

            @pl.when((j >= 1) & (j <= nt))
            def _():
                scatter(kn1, o).wait()

            @pl.when(j + 1 < nt)
            def _():
                idx_wait(j + 1, kn1)
                gather(kn1, o).start()

            @pl.when(j + 2 < nt)
            def _():
                idx_issue(j + 2, kn2)

            @pl.when(j < nt)
            def _():
                gather(k, b).wait()
                scatter(k, b).start(add=True)
        return carry

    lax.fori_loop(0, CH // 4, body, 0)
    plsc.subcore_barrier()
    pltpu.sync_copy(acc.at[pl.ds(s * RPT, RPT)], out_hbm.at[c, pl.ds(s * RPT, RPT)])


def _dense_body(nr, p, h, w1, g1, b1, w2, g2, b2, xo, pool):
    mask = lax.broadcasted_iota(jnp.int32, (NP, 1), 0) < N
    x = jnp.where(mask, p[0] + p[1] - h[...], 0.0)
    t = jnp.dot(x, w1[...], preferred_element_type=jnp.float32)
    mu = jnp.sum(t, axis=0, keepdims=True) * (1.0 / N)
    d = jnp.where(mask, t - mu, 0.0)
    var = jnp.sum(d * d, axis=0, keepdims=True) * (1.0 / N)
    y = g1[...] * d * jax.lax.rsqrt(var + 1e-5) + b1[...]
    y = jnp.where(mask, jnp.maximum(y, 0.0), 0.0)
    u = jnp.dot(y, w2[...], preferred_element_type=jnp.float32)
    mu2 = jnp.sum(u, axis=0, keepdims=True) * (1.0 / N)
    d2 = jnp.where(mask, u - mu2, 0.0)
    var2 = jnp.sum(d2 * d2, axis=0, keepdims=True) * (1.0 / N)
    z = g2[...] * d2 * jax.lax.rsqrt(var2 + 1e-5) + b2[...]
    z = jnp.where(mask, jnp.maximum(z, 0.0), 0.0)
    xo[...] = z[:nr]
    pool[...] = jnp.sum(z, axis=0, keepdims=True)


@functools.cache
def _make_dense(nr):
    return pl.pallas_call(
        functools.partial(_dense_body, nr),
        out_shape=(
            jax.ShapeDtypeStruct((nr, D), jnp.float32),
            jax.ShapeDtypeStruct((1, D), jnp.float32),
        ),
    )


def _layer(h_pad, edge_index, nr, W1, g1, b1, W2, bng, bnb):
    p = _make_sc_segment()(h_pad, edge_index)
    return _make_dense(nr)(p, h_pad,
                           W1, g1.reshape(1, D), b1.reshape(1, D),
                           W2, bng.reshape(1, D), bnb.reshape(1, D))


def kernel(h, edge_index, W1_0, g1_0, b1_0, W2_0, bng_0, bnb_0,
           W1_1, g1_1, b1_1, W2_1, bng_1, bnb_1):
    h_pad = jnp.pad(h, ((0, NP - N), (0, 0)))
    h1, p0 = _layer(h_pad, edge_index, NP, W1_0, g1_0, b1_0, W2_0, bng_0, bnb_0)
    h2, p1 = _layer(h1, edge_index, N, W1_1, g1_1, b1_1, W2_1, bng_1, bnb_1)
    return h2, jnp.concatenate([p0, p1], axis=1)


# speedup vs baseline: 3.9772x; 1.0827x over previous
import functools

import jax
import jax.numpy as jnp
from jax import lax
from jax.experimental import pallas as pl
from jax.experimental.pallas import tpu as pltpu
from jax.experimental.pallas import tpu_sc as plsc

N = 10000
E = 320000
D = 128

NW = 32
CK = 128
CH = 84

FULL = E // CK
NTLO = FULL // NW
NTREM = FULL % NW

SRPT = 632
SLAST = N - 15 * SRPT


NR = 3
NI = 4


@functools.cache
def _make_sc_segment():
    mesh = plsc.VectorSubcoreMesh(core_axis_name="c", subcore_axis_name="s")
    return pl.kernel(
        _sc_segment_body,
        out_type=jax.ShapeDtypeStruct((2, N, D), jnp.float32),
        mesh=mesh,
        scratch_types=[
            [pltpu.VMEM((2, CK), jnp.int32) for _ in range(NI)],
            [pltpu.VMEM((CK, D), jnp.float32) for _ in range(NR)],
            pltpu.VMEM_SHARED((N, D), jnp.float32),
            [pltpu.SemaphoreType.DMA for _ in range(NI)],
            [pltpu.SemaphoreType.DMA for _ in range(NR)],
            [pltpu.SemaphoreType.DMA for _ in range(NR)],
        ],
    )


def _sc_segment_body(h_hbm, eidx_hbm, out_hbm, idxs, rows, acc,
                     isems, gsems, ssems):
    c = lax.axis_index("c")
    s = lax.axis_index("s")
    wid = c * 16 + s
    nt = jnp.where(wid < NTREM, NTLO + 1, NTLO)

    def idx_copy(j, slot, row):
        return pltpu.make_async_copy(
            eidx_hbm.at[row, pl.ds(j * (NW * CK) + wid * CK, CK)],
            idxs[slot].at[row], isems[slot])

    def idx_issue(j, slot):
        idx_copy(j, slot, 0).start()
        idx_copy(j, slot, 1).start()

    def idx_wait(j, slot):
        idx_copy(j, slot, 0).wait()
        idx_copy(j, slot, 1).wait()

    def gather(islot, b):
        return pltpu.make_async_copy(
            h_hbm.at[idxs[islot].at[0]], rows[b], gsems[b])

    def scatter(islot, b):
        return pltpu.make_async_copy(
            rows[b], acc.at[idxs[islot].at[1]], ssems[b])

    idx_issue(0, 0)
    idx_issue(1, 1)
    idx_wait(0, 0)
    gather(0, 0).start()

    @pl.when(s < 15)
    def _():
        pltpu.sync_copy(h_hbm.at[pl.ds(s * SRPT, SRPT)],
                        acc.at[pl.ds(s * SRPT, SRPT)])

    @pl.when(s == 15)
    def _():
        pltpu.sync_copy(h_hbm.at[pl.ds(15 * SRPT, SLAST)],
                        acc.at[pl.ds(15 * SRPT, SLAST)])

    plsc.subcore_barrier()

    def steps(g, carry):
        for k in range(12):
            j = g * 12 + k

            @pl.when((j >= 2) & (j - 2 < nt))
            def _():
                scatter((k - 2) % 4, (k - 2) % 3).wait()

            @pl.when(j + 1 < nt)
            def _():
                idx_wait(j + 1, (k + 1) % 4)
                gather((k + 1) % 4, (k + 1) % 3).start()

            @pl.when(j + 2 < nt)
            def _():
                idx_issue(j + 2, (k + 2) % 4)

            @pl.when(j < nt)
            def _():
                gather(k % 4, k % 3).wait()
                scatter(k % 4, k % 3).start(add=True)
        return carry

    lax.fori_loop(0, CH // 12, steps, 0)
    plsc.subcore_barrier()

    @pl.when(s < 15)
    def _():
        pltpu.sync_copy(acc.at[pl.ds(s * SRPT, SRPT)],
                        out_hbm.at[c, pl.ds(s * SRPT, SRPT)])

    @pl.when(s == 15)
    def _():
        pltpu.sync_copy(acc.at[pl.ds(15 * SRPT, SLAST)],
                        out_hbm.at[c, pl.ds(15 * SRPT, SLAST)])


def _dense_body(p, h, w1, g1, b1, w2, g2, b2, xo, pool):
    x = p[0] + p[1] - h[...]
    t = jnp.dot(x, w1[...], preferred_element_type=jnp.float32)
    mu = jnp.sum(t, axis=0, keepdims=True) * (1.0 / N)
    d = t - mu
    var = jnp.sum(d * d, axis=0, keepdims=True) * (1.0 / N)
    y = g1[...] * d * jax.lax.rsqrt(var + 1e-5) + b1[...]
    y = jnp.maximum(y, 0.0)
    u = jnp.dot(y, w2[...], preferred_element_type=jnp.float32)
    mu2 = jnp.sum(u, axis=0, keepdims=True) * (1.0 / N)
    d2 = u - mu2
    var2 = jnp.sum(d2 * d2, axis=0, keepdims=True) * (1.0 / N)
    z = g2[...] * d2 * jax.lax.rsqrt(var2 + 1e-5) + b2[...]
    z = jnp.maximum(z, 0.0)
    xo[...] = z
    pool[...] = jnp.sum(z, axis=0, keepdims=True)


_dense = pl.pallas_call(
    _dense_body,
    out_shape=(
        jax.ShapeDtypeStruct((N, D), jnp.float32),
        jax.ShapeDtypeStruct((1, D), jnp.float32),
    ),
)


def _layer(h, edge_index, W1, g1, b1, W2, bng, bnb):
    p = _make_sc_segment()(h, edge_index)
    return _dense(p, h,
                  W1, g1.reshape(1, D), b1.reshape(1, D),
                  W2, bng.reshape(1, D), bnb.reshape(1, D))


def kernel(h, edge_index, W1_0, g1_0, b1_0, W2_0, bng_0, bnb_0,
           W1_1, g1_1, b1_1, W2_1, bng_1, bnb_1):
    h1, p0 = _layer(h, edge_index, W1_0, g1_0, b1_0, W2_0, bng_0, bnb_0)
    h2, p1 = _layer(h1, edge_index, W1_1, g1_1, b1_1, W2_1, bng_1, bnb_1)
    return h2, jnp.concatenate([p0, p1], axis=1)


# speedup vs baseline: 4.0813x; 1.0262x over previous
import functools

import jax
import jax.numpy as jnp
from jax import lax
from jax.experimental import pallas as pl
from jax.experimental.pallas import tpu as pltpu
from jax.experimental.pallas import tpu_sc as plsc

N = 10000
E = 320000
D = 128

NW = 32
CK = 128
CH = 84

FULL = E // CK
NTLO = FULL // NW
NTREM = FULL % NW

SRPT = 632
SLAST = N - 15 * SRPT


NR = 3
NI = 4


@functools.cache
def _make_sc_segment():
    mesh = plsc.VectorSubcoreMesh(core_axis_name="c", subcore_axis_name="s")
    return pl.kernel(
        _sc_segment_body,
        out_type=jax.ShapeDtypeStruct((2, N, D), jnp.float32),
        mesh=mesh,
        scratch_types=[
            [pltpu.VMEM((2, CK), jnp.int32) for _ in range(NI)],
            [pltpu.VMEM((CK, D), jnp.float32) for _ in range(NR)],
            pltpu.VMEM_SHARED((N, D), jnp.float32),
            [pltpu.SemaphoreType.DMA for _ in range(NI)],
            [pltpu.SemaphoreType.DMA for _ in range(NR)],
            [pltpu.SemaphoreType.DMA for _ in range(NR)],
        ],
    )


def _sc_segment_body(h_hbm, z_hbm, eidx_hbm, out_hbm, idxs, rows, acc,
                     isems, gsems, ssems):
    c = lax.axis_index("c")
    s = lax.axis_index("s")
    wid = c * 16 + s
    nt = jnp.where(wid < NTREM, NTLO + 1, NTLO)

    def idx_copy(j, slot, row):
        return pltpu.make_async_copy(
            eidx_hbm.at[row, pl.ds(j * (NW * CK) + wid * CK, CK)],
            idxs[slot].at[row], isems[slot])

    def idx_issue(j, slot):
        idx_copy(j, slot, 0).start()
        idx_copy(j, slot, 1).start()

    def idx_wait(j, slot):
        idx_copy(j, slot, 0).wait()
        idx_copy(j, slot, 1).wait()

    def gather(islot, b):
        return pltpu.make_async_copy(
            h_hbm.at[idxs[islot].at[0]], rows[b], gsems[b])

    def scatter(islot, b):
        return pltpu.make_async_copy(
            rows[b], acc.at[idxs[islot].at[1]], ssems[b])

    idx_issue(0, 0)
    idx_issue(1, 1)
    idx_wait(0, 0)
    gather(0, 0).start()

    for core, src in ((0, h_hbm), (1, z_hbm)):
        @pl.when((c == core) & (s < 15))
        def _():
            pltpu.sync_copy(src.at[pl.ds(s * SRPT, SRPT)],
                            acc.at[pl.ds(s * SRPT, SRPT)])

        @pl.when((c == core) & (s == 15))
        def _():
            pltpu.sync_copy(src.at[pl.ds(15 * SRPT, SLAST)],
                            acc.at[pl.ds(15 * SRPT, SLAST)])

    plsc.subcore_barrier()

    def steps(g, carry):
        for k in range(12):
            j = g * 12 + k

            @pl.when((j >= 2) & (j - 2 < nt))
            def _():
                scatter((k - 2) % 4, (k - 2) % 3).wait()

            @pl.when(j + 1 < nt)
            def _():
                idx_wait(j + 1, (k + 1) % 4)
                gather((k + 1) % 4, (k + 1) % 3).start()

            @pl.when(j + 2 < nt)
            def _():
                idx_issue(j + 2, (k + 2) % 4)

            @pl.when(j < nt)
            def _():
                gather(k % 4, k % 3).wait()
                scatter(k % 4, k % 3).start(add=True)
        return carry

    lax.fori_loop(0, CH // 12, steps, 0)
    plsc.subcore_barrier()

    @pl.when(s < 15)
    def _():
        pltpu.sync_copy(acc.at[pl.ds(s * SRPT, SRPT)],
                        out_hbm.at[c, pl.ds(s * SRPT, SRPT)])

    @pl.when(s == 15)
    def _():
        pltpu.sync_copy(acc.at[pl.ds(15 * SRPT, SLAST)],
                        out_hbm.at[c, pl.ds(15 * SRPT, SLAST)])


def _dense_body(p, w1, g1, b1, w2, g2, b2, xo, pool):
    x = p[0] + p[1]
    t = jnp.dot(x, w1[...], preferred_element_type=jnp.float32)
    mu = jnp.sum(t, axis=0, keepdims=True) * (1.0 / N)
    var = jnp.sum(t * t, axis=0, keepdims=True) * (1.0 / N) - mu * mu
    y = g1[...] * (t - mu) * jax.lax.rsqrt(var + 1e-5) + b1[...]
    y = jnp.maximum(y, 0.0)
    u = jnp.dot(y, w2[...], preferred_element_type=jnp.float32)
    mu2 = jnp.sum(u, axis=0, keepdims=True) * (1.0 / N)
    var2 = jnp.sum(u * u, axis=0, keepdims=True) * (1.0 / N) - mu2 * mu2
    z = g2[...] * (u - mu2) * jax.lax.rsqrt(var2 + 1e-5) + b2[...]
    z = jnp.maximum(z, 0.0)
    xo[...] = z
    pool[...] = jnp.sum(z, axis=0, keepdims=True)


_dense = pl.pallas_call(
    _dense_body,
    out_shape=(
        jax.ShapeDtypeStruct((N, D), jnp.float32),
        jax.ShapeDtypeStruct((1, D), jnp.float32),
    ),
)


def _layer(h, zeros, edge_index, W1, g1, b1, W2, bng, bnb):
    p = _make_sc_segment()(h, zeros, edge_index)
    return _dense(p,
                  W1, g1.reshape(1, D), b1.reshape(1, D),
                  W2, bng.reshape(1, D), bnb.reshape(1, D))


def kernel(h, edge_index, W1_0, g1_0, b1_0, W2_0, bng_0, bnb_0,
           W1_1, g1_1, b1_1, W2_1, bng_1, bnb_1):
    zeros = jnp.zeros((N, D), jnp.float32)
    h1, p0 = _layer(h, zeros, edge_index, W1_0, g1_0, b1_0, W2_0, bng_0, bnb_0)
    h2, p1 = _layer(h1, zeros, edge_index, W1_1, g1_1, b1_1, W2_1, bng_1, bnb_1)
    return h2, jnp.concatenate([p0, p1], axis=1)


# speedup vs baseline: 4.1212x; 1.0098x over previous
import functools

import jax
import jax.numpy as jnp
from jax import lax
from jax.experimental import pallas as pl
from jax.experimental.pallas import tpu as pltpu
from jax.experimental.pallas import tpu_sc as plsc

N = 10000
E = 320000
D = 128

NW = 32
CK = 128
CH = 84

FULL = E // CK
NTLO = FULL // NW
NTREM = FULL % NW

SRPT = 632
SLAST = N - 15 * SRPT


NR = 3
NI = 4


@functools.cache
def _make_sc_segment():
    mesh = plsc.VectorSubcoreMesh(core_axis_name="c", subcore_axis_name="s")
    return pl.kernel(
        _sc_segment_body,
        out_type=jax.ShapeDtypeStruct((2, N, D), jnp.float32),
        mesh=mesh,
        scratch_types=[
            [pltpu.VMEM((2, CK), jnp.int32) for _ in range(NI)],
            [pltpu.VMEM((CK, D), jnp.float32) for _ in range(NR)],
            pltpu.VMEM_SHARED((N, D), jnp.float32),
            [pltpu.SemaphoreType.DMA for _ in range(NI)],
            [pltpu.SemaphoreType.DMA for _ in range(NR)],
            [pltpu.SemaphoreType.DMA for _ in range(NR)],
        ],
    )


def _sc_segment_body(h_hbm, eidx_hbm, out_hbm, idxs, rows, acc,
                     isems, gsems, ssems):
    c = lax.axis_index("c")
    s = lax.axis_index("s")
    wid = c * 16 + s
    nt = jnp.where(wid < NTREM, NTLO + 1, NTLO)

    def idx_copy(j, slot, row):
        return pltpu.make_async_copy(
            eidx_hbm.at[row, pl.ds(j * (NW * CK) + wid * CK, CK)],
            idxs[slot].at[row], isems[slot])

    def idx_issue(j, slot):
        idx_copy(j, slot, 0).start()
        idx_copy(j, slot, 1).start()

    def idx_wait(j, slot):
        idx_copy(j, slot, 0).wait()
        idx_copy(j, slot, 1).wait()

    def gather(islot, b):
        return pltpu.make_async_copy(
            h_hbm.at[idxs[islot].at[0]], rows[b], gsems[b])

    def scatter(islot, b):
        return pltpu.make_async_copy(
            rows[b], acc.at[idxs[islot].at[1]], ssems[b])

    idx_issue(0, 0)
    idx_issue(1, 1)
    idx_wait(0, 0)
    gather(0, 0).start()

    @pl.when(c == 0)
    def _():
        @pl.when(s < 15)
        def _():
            pltpu.sync_copy(h_hbm.at[pl.ds(s * SRPT, SRPT)],
                            acc.at[pl.ds(s * SRPT, SRPT)])

        @pl.when(s == 15)
        def _():
            pltpu.sync_copy(h_hbm.at[pl.ds(15 * SRPT, SLAST)],
                            acc.at[pl.ds(15 * SRPT, SLAST)])

    @pl.when(c == 1)
    def _():
        def zrow(i, carry):
            for q in range(D // 16):
                rows[2][i, pl.ds(q * 16, 16)] = jnp.zeros((16,), jnp.float32)
            return carry

        lax.fori_loop(0, CK, zrow, 0)
        base = s * SRPT
        for t in range(4):
            pltpu.sync_copy(rows[2], acc.at[pl.ds(base + t * CK, CK)])

        @pl.when(s < 15)
        def _():
            pltpu.sync_copy(rows[2].at[pl.ds(0, SRPT - 4 * CK)],
                            acc.at[pl.ds(base + 4 * CK, SRPT - 4 * CK)])

        @pl.when(s == 15)
        def _():
            pltpu.sync_copy(rows[2].at[pl.ds(0, SLAST - 4 * CK)],
                            acc.at[pl.ds(base + 4 * CK, SLAST - 4 * CK)])

    plsc.subcore_barrier()

    def steps(g, carry):
        for k in range(12):
            j = g * 12 + k

            @pl.when((j >= 2) & (j - 2 < nt))
            def _():
                scatter((k - 2) % 4, (k - 2) % 3).wait()

            @pl.when(j + 1 < nt)
            def _():
                idx_wait(j + 1, (k + 1) % 4)
                gather((k + 1) % 4, (k + 1) % 3).start()

            @pl.when(j + 2 < nt)
            def _():
                idx_issue(j + 2, (k + 2) % 4)

            @pl.when(j < nt)
            def _():
                gather(k % 4, k % 3).wait()
                scatter(k % 4, k % 3).start(add=True)
        return carry

    lax.fori_loop(0, CH // 12, steps, 0)
    plsc.subcore_barrier()

    @pl.when(s < 15)
    def _():
        pltpu.sync_copy(acc.at[pl.ds(s * SRPT, SRPT)],
                        out_hbm.at[c, pl.ds(s * SRPT, SRPT)])

    @pl.when(s == 15)
    def _():
        pltpu.sync_copy(acc.at[pl.ds(15 * SRPT, SLAST)],
                        out_hbm.at[c, pl.ds(15 * SRPT, SLAST)])


def _dense_body(p, w1, g1, b1, w2, g2, b2, xo, pool):
    x = p[0] + p[1]
    t = jnp.dot(x, w1[...], preferred_element_type=jnp.float32)
    mu = jnp.sum(t, axis=0, keepdims=True) * (1.0 / N)
    var = jnp.sum(t * t, axis=0, keepdims=True) * (1.0 / N) - mu * mu
    y = g1[...] * (t - mu) * jax.lax.rsqrt(var + 1e-5) + b1[...]
    y = jnp.maximum(y, 0.0)
    u = jnp.dot(y, w2[...], preferred_element_type=jnp.float32)
    mu2 = jnp.sum(u, axis=0, keepdims=True) * (1.0 / N)
    var2 = jnp.sum(u * u, axis=0, keepdims=True) * (1.0 / N) - mu2 * mu2
    z = g2[...] * (u - mu2) * jax.lax.rsqrt(var2 + 1e-5) + b2[...]
    z = jnp.maximum(z, 0.0)
    xo[...] = z
    pool[...] = jnp.sum(z, axis=0, keepdims=True)


_dense = pl.pallas_call(
    _dense_body,
    out_shape=(
        jax.ShapeDtypeStruct((N, D), jnp.float32),
        jax.ShapeDtypeStruct((1, D), jnp.float32),
    ),
)


def _layer(h, edge_index, W1, g1, b1, W2, bng, bnb):
    p = _make_sc_segment()(h, edge_index)
    return _dense(p,
                  W1, g1.reshape(1, D), b1.reshape(1, D),
                  W2, bng.reshape(1, D), bnb.reshape(1, D))


def kernel(h, edge_index, W1_0, g1_0, b1_0, W2_0, bng_0, bnb_0,
           W1_1, g1_1, b1_1, W2_1, bng_1, bnb_1):
    h1, p0 = _layer(h, edge_index, W1_0, g1_0, b1_0, W2_0, bng_0, bnb_0)
    h2, p1 = _layer(h1, edge_index, W1_1, g1_1, b1_1, W2_1, bng_1, bnb_1)
    return h2, jnp.concatenate([p0, p1], axis=1)
